# baseline (speedup 1.0000x reference)
import jax
import jax.numpy as jnp
from jax.experimental import pallas as pl
from jax.experimental.pallas import tpu as pltpu


def _pick_bm(n: int) -> int:
    for bm in (200, 400, 1000, 40, 8):
        if n % bm == 0 and bm <= n:
            return bm
    return n


def _dot_bf16(a_bf16, b_bf16):
    return jax.lax.dot_general(
        a_bf16, b_bf16, (((1,), (0,)), ((), ())),
        preferred_element_type=jnp.float32)


def kernel(x, phi0, phi1, W, kernel):
    n, d_in = x.shape
    d_out = W.shape[1]
    bm = _pick_bm(n)
    nb = n // bm

    def body(x_ref, w_ref, k_ref, phi0_ref, phi1_ref, out_ref,
             buf, xpbuf, tbuf, sems):
        s = pl.program_id(0)

        def issue(gs):
            slot = jax.lax.rem(gs, 2)

            @pl.when(gs < nb)
            def _():
                pltpu.make_async_copy(
                    phi0_ref.at[pl.ds(gs * bm, bm), :],
                    buf.at[slot], sems.at[slot]).start()

            @pl.when(jnp.logical_and(gs >= nb, gs < 2 * nb))
            def _():
                pltpu.make_async_copy(
                    phi1_ref.at[pl.ds((gs - nb) * bm, bm), :],
                    buf.at[slot], sems.at[slot]).start()

        @pl.when(s == 0)
        def _():
            issue(0)

        issue(s + 1)

        @pl.when(s == 0)
        def _():
            xpbuf[...] = _dot_bf16(
                x_ref[...].astype(jnp.bfloat16),
                w_ref[...].astype(jnp.bfloat16)).astype(jnp.bfloat16)

        slot = jax.lax.rem(s, 2)
        pltpu.make_async_copy(
            phi0_ref.at[pl.ds(0, bm), :], buf.at[slot], sems.at[slot]).wait()

        vb = buf[slot].astype(jnp.bfloat16)

        @pl.when(s < nb)
        def _():
            t = _dot_bf16(vb, xpbuf[...])
            tbuf[pl.ds(s * bm, bm), :] = (k_ref[...] * t).astype(jnp.bfloat16)

        @pl.when(s >= nb)
        def _():
            o = _dot_bf16(vb, tbuf[...])
            out_ref[...] = jnp.maximum(o, 0.0)

    nb_minus_1 = nb - 1
    out = pl.pallas_call(
        body,
        grid=(2 * nb,),
        in_specs=[
            pl.BlockSpec((n, d_in), lambda s: (0, 0)),
            pl.BlockSpec((d_in, d_out), lambda s: (0, 0)),
            pl.BlockSpec((bm, 1), lambda s: (jnp.minimum(s, nb_minus_1), 0)),
            pl.BlockSpec(memory_space=pl.ANY),
            pl.BlockSpec(memory_space=pl.ANY),
        ],
        out_specs=pl.BlockSpec(
            (bm, d_out), lambda s: (jnp.maximum(s - nb, 0), 0)),
        out_shape=jax.ShapeDtypeStruct((n, d_out), jnp.float32),
        scratch_shapes=[
            pltpu.VMEM((2, bm, n), jnp.float32),
            pltpu.VMEM((n, d_out), jnp.bfloat16),
            pltpu.VMEM((n, d_out), jnp.bfloat16),
            pltpu.SemaphoreType.DMA((2,)),
        ],
    )(x, W, kernel, phi0, phi1)

    return out


# speedup vs baseline: 1.0329x; 1.0329x over previous
import jax
import jax.numpy as jnp
from jax.experimental import pallas as pl


def _pick_bm(n: int) -> int:
    for bm in (400, 200, 1000, 40, 8):
        if n % bm == 0 and bm <= n:
            return bm
    return n


def _dot(a, b):
    return jax.lax.dot_general(
        a, b, (((1,), (0,)), ((), ())),
        preferred_element_type=jnp.float32,
        precision=jax.lax.Precision.DEFAULT)


def _xw_kernel(x_ref, w_ref, out_ref):
    out_ref[...] = _dot(x_ref[...], w_ref[...])


def _phi_scale_kernel(phi_ref, v_ref, k_ref, out_ref):
    out_ref[...] = k_ref[...] * _dot(phi_ref[...], v_ref[...])


def _phi_relu_kernel(phi_ref, v_ref, out_ref):
    out_ref[...] = jnp.maximum(_dot(phi_ref[...], v_ref[...]), 0.0)


def kernel(x, phi0, phi1, W, kernel):
    n, d_in = x.shape
    d_out = W.shape[1]
    bm = _pick_bm(n)
    grid = (n // bm,)

    xp = pl.pallas_call(
        _xw_kernel,
        out_shape=jax.ShapeDtypeStruct((n, d_out), jnp.float32),
    )(x, W)

    t = pl.pallas_call(
        _phi_scale_kernel,
        grid=grid,
        in_specs=[
            pl.BlockSpec((bm, n), lambda i: (i, 0)),
            pl.BlockSpec((n, d_out), lambda i: (0, 0)),
            pl.BlockSpec((bm, 1), lambda i: (i, 0)),
        ],
        out_specs=pl.BlockSpec((bm, d_out), lambda i: (i, 0)),
        out_shape=jax.ShapeDtypeStruct((n, d_out), jnp.float32),
    )(phi0, xp, kernel)

    out = pl.pallas_call(
        _phi_relu_kernel,
        grid=grid,
        in_specs=[
            pl.BlockSpec((bm, n), lambda i: (i, 0)),
            pl.BlockSpec((n, d_out), lambda i: (0, 0)),
        ],
        out_specs=pl.BlockSpec((bm, d_out), lambda i: (i, 0)),
        out_shape=jax.ShapeDtypeStruct((n, d_out), jnp.float32),
    )(phi1, t)

    return out


# speedup vs baseline: 1.0840x; 1.0494x over previous
import jax
import jax.numpy as jnp
from jax.experimental import pallas as pl
from jax.experimental.pallas import tpu as pltpu


def _pick_bm(n: int) -> int:
    for bm in (200, 400, 1000, 40, 8):
        if n % bm == 0 and bm <= n:
            return bm
    return n


def _dot(a, b):
    return jax.lax.dot_general(
        a, b, (((1,), (0,)), ((), ())),
        preferred_element_type=jnp.float32,
        precision=jax.lax.Precision.DEFAULT)


def kernel(x, phi0, phi1, W, kernel):
    n, d_in = x.shape
    d_out = W.shape[1]
    bm = _pick_bm(n)
    nb = n // bm

    def body(x_ref, w_ref, k_ref, phi0_ref, phi1_ref, out_ref,
             buf, xpbuf, tbuf, sems):
        s = pl.program_id(0)

        def issue(gs):
            slot = jax.lax.rem(gs, 2)

            @pl.when(gs < nb)
            def _():
                pltpu.make_async_copy(
                    phi0_ref.at[pl.ds(gs * bm, bm), :],
                    buf.at[slot], sems.at[slot]).start()

            @pl.when(jnp.logical_and(gs >= nb, gs < 2 * nb))
            def _():
                pltpu.make_async_copy(
                    phi1_ref.at[pl.ds((gs - nb) * bm, bm), :],
                    buf.at[slot], sems.at[slot]).start()

        @pl.when(s == 0)
        def _():
            issue(0)

        issue(s + 1)

        @pl.when(s == 0)
        def _():
            xpbuf[...] = _dot(x_ref[...], w_ref[...])

        slot = jax.lax.rem(s, 2)
        pltpu.make_async_copy(
            phi0_ref.at[pl.ds(0, bm), :], buf.at[slot], sems.at[slot]).wait()

        @pl.when(s < nb)
        def _():
            t = _dot(buf[slot], xpbuf[...])
            tbuf[pl.ds(s * bm, bm), :] = k_ref[...] * t

        @pl.when(s >= nb)
        def _():
            o = _dot(buf[slot], tbuf[...])
            out_ref[...] = jnp.maximum(o, 0.0)

    nb_minus_1 = nb - 1
    out = pl.pallas_call(
        body,
        grid=(2 * nb,),
        in_specs=[
            pl.BlockSpec((n, d_in), lambda s: (0, 0)),
            pl.BlockSpec((d_in, d_out), lambda s: (0, 0)),
            pl.BlockSpec((bm, 1), lambda s: (jnp.minimum(s, nb_minus_1), 0)),
            pl.BlockSpec(memory_space=pl.ANY),
            pl.BlockSpec(memory_space=pl.ANY),
        ],
        out_specs=pl.BlockSpec(
            (bm, d_out), lambda s: (jnp.maximum(s - nb, 0), 0)),
        out_shape=jax.ShapeDtypeStruct((n, d_out), jnp.float32),
        scratch_shapes=[
            pltpu.VMEM((2, bm, n), jnp.float32),
            pltpu.VMEM((n, d_out), jnp.float32),
            pltpu.VMEM((n, d_out), jnp.float32),
            pltpu.SemaphoreType.DMA((2,)),
        ],
    )(x, W, kernel, phi0, phi1)

    return out


# speedup vs baseline: 1.1081x; 1.0223x over previous
import jax
import jax.numpy as jnp
from jax.experimental import pallas as pl
from jax.experimental.pallas import tpu as pltpu


def _pick_bm(n: int) -> int:
    for bm in (200, 400, 1000, 40, 8):
        if n % bm == 0 and bm <= n:
            return bm
    return n


def _dot(a, b):
    return jax.lax.dot_general(
        a, b, (((1,), (0,)), ((), ())),
        preferred_element_type=jnp.float32,
        precision=jax.lax.Precision.DEFAULT)


def kernel(x, phi0, phi1, W, kernel):
    n, d_in = x.shape
    d_out = W.shape[1]
    bm = _pick_bm(n)
    nb = n // bm

    nslots = 3

    def body(x_ref, w_ref, k_ref, phi0_ref, phi1_ref, out_ref,
             buf, xpbuf, tbuf, sems):
        s = pl.program_id(0)

        def issue(gs):
            slot = jax.lax.rem(gs, nslots)

            @pl.when(gs < nb)
            def _():
                pltpu.make_async_copy(
                    phi0_ref.at[pl.ds(gs * bm, bm), :],
                    buf.at[slot], sems.at[slot]).start()

            @pl.when(jnp.logical_and(gs >= nb, gs < 2 * nb))
            def _():
                pltpu.make_async_copy(
                    phi1_ref.at[pl.ds((gs - nb) * bm, bm), :],
                    buf.at[slot], sems.at[slot]).start()

        @pl.when(s == 0)
        def _():
            issue(0)
            issue(1)

        issue(s + 2)

        @pl.when(s == 0)
        def _():
            xpbuf[...] = _dot(x_ref[...], w_ref[...])

        slot = jax.lax.rem(s, nslots)
        pltpu.make_async_copy(
            phi0_ref.at[pl.ds(0, bm), :], buf.at[slot], sems.at[slot]).wait()

        @pl.when(s < nb)
        def _():
            t = _dot(buf[slot], xpbuf[...])
            tbuf[pl.ds(s * bm, bm), :] = k_ref[...] * t

        @pl.when(s >= nb)
        def _():
            o = _dot(buf[slot], tbuf[...])
            out_ref[...] = jnp.maximum(o, 0.0)

    nb_minus_1 = nb - 1
    out = pl.pallas_call(
        body,
        grid=(2 * nb,),
        in_specs=[
            pl.BlockSpec((n, d_in), lambda s: (0, 0)),
            pl.BlockSpec((d_in, d_out), lambda s: (0, 0)),
            pl.BlockSpec((bm, 1), lambda s: (jnp.minimum(s, nb_minus_1), 0)),
            pl.BlockSpec(memory_space=pl.ANY),
            pl.BlockSpec(memory_space=pl.ANY),
        ],
        out_specs=pl.BlockSpec(
            (bm, d_out), lambda s: (jnp.maximum(s - nb, 0), 0)),
        out_shape=jax.ShapeDtypeStruct((n, d_out), jnp.float32),
        scratch_shapes=[
            pltpu.VMEM((3, bm, n), jnp.float32),
            pltpu.VMEM((n, d_out), jnp.float32),
            pltpu.VMEM((n, d_out), jnp.float32),
            pltpu.SemaphoreType.DMA((3,)),
        ],
    )(x, W, kernel, phi0, phi1)

    return out
